# baseline (speedup 1.0000x reference)
import jax
import jax.numpy as jnp
from jax.experimental import pallas as pl
from jax.experimental.pallas import tpu as pltpu

N = 100000
D = 768
A = 64
H = 256
K = 16

BR = 4096
NB = 25
SR = BR // 128
RPAD = NB * SR
NEG = -1e30


def _fused_kernel(x_ref, w1_ref, b1_ref, w2_ref, x_hbm,
                  wc1_ref, bc1_ref, wc2_ref, bc2_ref,
                  logit_ref, emb_ref, fw_ref, idx_ref,
                  blk_ref, cv_ref, ci_ref, xk_ref, sem):
    i = pl.program_id(0)
    a = jnp.tanh(
        jnp.dot(x_ref[...], w1_ref[...], preferred_element_type=jnp.float32)
        + b1_ref[...]
    )
    s = jnp.dot(a, w2_ref[...], preferred_element_type=jnp.float32)
    s2 = s.reshape(SR, 128)
    flat = (
        i * BR
        + jax.lax.broadcasted_iota(jnp.int32, (SR, 128), 0) * 128
        + jax.lax.broadcasted_iota(jnp.int32, (SR, 128), 1)
    )
    blk_ref[...] = jnp.where(flat < N, s2, NEG)

    fw_ref[pl.ds(i * SR, SR), :] = jnp.zeros((SR, 128), jnp.float32)

    lane = jax.lax.broadcasted_iota(jnp.int32, (1, 128), 1)
    lane_k = jax.lax.broadcasted_iota(jnp.int32, (1, K), 1)
    row_iota = jax.lax.iota(jnp.int32, SR)
    rm = jnp.max(blk_ref[...], axis=1)
    cvrow = jnp.zeros((1, K), jnp.float32)
    cirow = jnp.zeros((1, K), jnp.int32)
    for j in range(K):
        r = jnp.argmax(rm).astype(jnp.int32)
        rowv = blk_ref[pl.ds(r, 1), :]
        c = jnp.argmax(rowv[0, :]).astype(jnp.int32)
        v = jnp.max(rowv)
        cvrow = jnp.where(lane_k == j, v, cvrow)
        cirow = jnp.where(lane_k == j, i * BR + r * 128 + c, cirow)
        newrow = jnp.where(lane == c, NEG, rowv)
        blk_ref[pl.ds(r, 1), :] = newrow
        rm = jnp.where(row_iota == r, jnp.max(newrow), rm)
    cv_ref[pl.ds(i, 1), :] = cvrow
    ci_ref[pl.ds(i, 1), :] = cirow

    @pl.when(i == NB - 1)
    def _finish():
        row_iota2 = jax.lax.iota(jnp.int32, NB)
        rm2 = jnp.max(cv_ref[...], axis=1)
        vals = []
        idxs = []
        copies = []
        for j in range(K):
            r = jnp.argmax(rm2).astype(jnp.int32)
            rowv = cv_ref[pl.ds(r, 1), :]
            c = jnp.argmax(rowv[0, :]).astype(jnp.int32)
            v = jnp.max(rowv)
            rowi = ci_ref[pl.ds(r, 1), :]
            fidx = jnp.sum(jnp.where(lane_k == c, rowi, 0))
            vals.append(v)
            idxs.append(fidx)
            newrow = jnp.where(lane_k == c, NEG, rowv)
            cv_ref[pl.ds(r, 1), :] = newrow
            rm2 = jnp.where(row_iota2 == r, jnp.max(newrow), rm2)
            cp = pltpu.make_async_copy(
                x_hbm.at[pl.ds(fidx, 1), :], xk_ref.at[pl.ds(j, 1), :],
                sem)
            cp.start()
            copies.append(cp)

        exps = [jnp.exp(v - vals[0]) for v in vals]
        denom = exps[0]
        for e in exps[1:]:
            denom = denom + e
        ws = [e / denom for e in exps]

        wrow = jnp.zeros((1, K), jnp.float32)
        irow = jnp.zeros((1, K), jnp.int32)
        for j in range(K):
            wrow = jnp.where(lane_k == j, ws[j], wrow)
            irow = jnp.where(lane_k == j, idxs[j], irow)
        idx_ref[...] = irow

        for cp in copies:
            cp.wait()

        emb = jnp.dot(wrow, xk_ref[...],
                      preferred_element_type=jnp.float32)
        emb_ref[...] = emb
        h = jnp.maximum(
            jnp.dot(emb, wc1_ref[...], preferred_element_type=jnp.float32)
            + bc1_ref[...], 0.0)
        logit_ref[...] = (
            jnp.dot(h, wc2_ref[...], preferred_element_type=jnp.float32)
            + bc2_ref[...])

        for j in range(K):
            fr = idxs[j] // 128
            fc = idxs[j] % 128
            cur = fw_ref[pl.ds(fr, 1), :]
            fw_ref[pl.ds(fr, 1), :] = jnp.where(lane == fc, ws[j], cur)


@jax.jit
def kernel(x, W1, b1, W2, b2, Wc1, bc1, Wc2, bc2):
    logit2d, emb2d, fw2d, idx2d = pl.pallas_call(
        _fused_kernel,
        grid=(NB,),
        in_specs=[
            pl.BlockSpec((BR, D), lambda i: (i, 0)),
            pl.BlockSpec((D, A), lambda i: (0, 0)),
            pl.BlockSpec((1, A), lambda i: (0, 0)),
            pl.BlockSpec((A, 1), lambda i: (0, 0)),
            pl.BlockSpec(memory_space=pl.ANY),
            pl.BlockSpec((D, H), lambda i: (0, 0)),
            pl.BlockSpec((1, H), lambda i: (0, 0)),
            pl.BlockSpec((H, 1), lambda i: (0, 0)),
            pl.BlockSpec((1, 1), lambda i: (0, 0)),
        ],
        out_specs=(
            pl.BlockSpec((1, 1), lambda i: (0, 0)),
            pl.BlockSpec((1, D), lambda i: (0, 0)),
            pl.BlockSpec((RPAD, 128), lambda i: (0, 0)),
            pl.BlockSpec((1, K), lambda i: (0, 0)),
        ),
        out_shape=(
            jax.ShapeDtypeStruct((1, 1), jnp.float32),
            jax.ShapeDtypeStruct((1, D), jnp.float32),
            jax.ShapeDtypeStruct((RPAD, 128), jnp.float32),
            jax.ShapeDtypeStruct((1, K), jnp.int32),
        ),
        scratch_shapes=[
            pltpu.VMEM((SR, 128), jnp.float32),
            pltpu.VMEM((NB, K), jnp.float32),
            pltpu.VMEM((NB, K), jnp.int32),
            pltpu.VMEM((K, D), jnp.float32),
            pltpu.SemaphoreType.DMA,
        ],
        compiler_params=pltpu.CompilerParams(
            dimension_semantics=("arbitrary",)),
    )(x, W1, b1.reshape(1, A), W2, x,
      Wc1, bc1.reshape(1, H), Wc2, bc2.reshape(1, 1))

    logit = logit2d.reshape(())
    slide_embedding = emb2d.reshape(D)
    full_weights = fw2d.reshape(RPAD * 128)[:N]
    topk_idx = idx2d.reshape(K)
    return (logit, slide_embedding, full_weights, topk_idx)


# speedup vs baseline: 1.9684x; 1.9684x over previous
import jax
import jax.numpy as jnp
from jax.experimental import pallas as pl
from jax.experimental.pallas import tpu as pltpu

N = 100000
D = 768
A = 64
H = 256
K = 16

BR = 4096
NB = 25
SR = BR // 128
RPAD = NB * SR
NEG = -1e30
IMAX = 2147483647


def _fused_kernel(x_ref, w1_ref, b1_ref, w2_ref, x_hbm,
                  wc1_ref, bc1_ref, wc2_ref, bc2_ref,
                  logit_ref, emb_ref, fw_ref, idx_ref,
                  mv_ref, mi_ref, xk_ref, sem):
    i = pl.program_id(0)
    a = jnp.tanh(
        jnp.dot(x_ref[...], w1_ref[...], preferred_element_type=jnp.float32)
        + b1_ref[...]
    )
    s = jnp.dot(a, w2_ref[...], preferred_element_type=jnp.float32)
    s2 = s.reshape(SR, 128)
    flat = (
        i * BR
        + jax.lax.broadcasted_iota(jnp.int32, (SR, 128), 0) * 128
        + jax.lax.broadcasted_iota(jnp.int32, (SR, 128), 1)
    )
    s2 = jnp.where(flat < N, s2, NEG)

    fw_ref[pl.ds(i * SR, SR), :] = jnp.zeros((SR, 128), jnp.float32)

    @pl.when(i == 0)
    def _init():
        mv_ref[...] = jnp.full((K, 128), NEG, jnp.float32)
        mi_ref[...] = jnp.zeros((K, 128), jnp.int32)

    cv = jnp.concatenate([mv_ref[...], s2], axis=0)
    ci = jnp.concatenate([mi_ref[...], flat], axis=0)
    mrows_v = []
    mrows_i = []
    for j in range(K):
        mx = jnp.max(cv, axis=0, keepdims=True)
        eq = cv == mx
        mrows_v.append(mx)
        mrows_i.append(jnp.min(jnp.where(eq, ci, IMAX), axis=0,
                               keepdims=True))
        cv = jnp.where(eq, NEG, cv)
    mv_ref[...] = jnp.concatenate(mrows_v, axis=0)
    mi_ref[...] = jnp.concatenate(mrows_i, axis=0)

    @pl.when(i == NB - 1)
    def _finish():
        lane = jax.lax.broadcasted_iota(jnp.int32, (1, 128), 1)
        lane_k = jax.lax.broadcasted_iota(jnp.int32, (1, K), 1)
        vals = []
        idxs = []
        copies = []
        for j in range(K):
            top = mv_ref[pl.ds(0, 1), :]
            c = jnp.argmax(top[0, :]).astype(jnp.int32)
            v = jnp.max(top)
            topi = mi_ref[pl.ds(0, 1), :]
            fidx = jnp.sum(jnp.where(lane == c, topi, 0))
            vals.append(v)
            idxs.append(fidx)
            shv = jnp.concatenate(
                [mv_ref[pl.ds(1, K - 1), :],
                 jnp.full((1, 128), NEG, jnp.float32)], axis=0)
            shi = jnp.concatenate(
                [mi_ref[pl.ds(1, K - 1), :],
                 jnp.zeros((1, 128), jnp.int32)], axis=0)
            mv_ref[...] = jnp.where(lane == c, shv, mv_ref[...])
            mi_ref[...] = jnp.where(lane == c, shi, mi_ref[...])
            cp = pltpu.make_async_copy(
                x_hbm.at[pl.ds(fidx, 1), :], xk_ref.at[pl.ds(j, 1), :],
                sem)
            cp.start()
            copies.append(cp)

        exps = [jnp.exp(v - vals[0]) for v in vals]
        denom = exps[0]
        for e in exps[1:]:
            denom = denom + e
        ws = [e / denom for e in exps]

        wrow = jnp.zeros((1, K), jnp.float32)
        irow = jnp.zeros((1, K), jnp.int32)
        for j in range(K):
            wrow = jnp.where(lane_k == j, ws[j], wrow)
            irow = jnp.where(lane_k == j, idxs[j], irow)
        idx_ref[...] = irow

        for cp in copies:
            cp.wait()

        emb = jnp.dot(wrow, xk_ref[...],
                      preferred_element_type=jnp.float32)
        emb_ref[...] = emb
        h = jnp.maximum(
            jnp.dot(emb, wc1_ref[...], preferred_element_type=jnp.float32)
            + bc1_ref[...], 0.0)
        logit_ref[...] = (
            jnp.dot(h, wc2_ref[...], preferred_element_type=jnp.float32)
            + bc2_ref[...])

        for j in range(K):
            fr = idxs[j] // 128
            fc = idxs[j] % 128
            cur = fw_ref[pl.ds(fr, 1), :]
            fw_ref[pl.ds(fr, 1), :] = jnp.where(lane == fc, ws[j], cur)


@jax.jit
def kernel(x, W1, b1, W2, b2, Wc1, bc1, Wc2, bc2):
    logit2d, emb2d, fw2d, idx2d = pl.pallas_call(
        _fused_kernel,
        grid=(NB,),
        in_specs=[
            pl.BlockSpec((BR, D), lambda i: (i, 0)),
            pl.BlockSpec((D, A), lambda i: (0, 0)),
            pl.BlockSpec((1, A), lambda i: (0, 0)),
            pl.BlockSpec((A, 1), lambda i: (0, 0)),
            pl.BlockSpec(memory_space=pl.ANY),
            pl.BlockSpec((D, H), lambda i: (0, 0)),
            pl.BlockSpec((1, H), lambda i: (0, 0)),
            pl.BlockSpec((H, 1), lambda i: (0, 0)),
            pl.BlockSpec((1, 1), lambda i: (0, 0)),
        ],
        out_specs=(
            pl.BlockSpec((1, 1), lambda i: (0, 0)),
            pl.BlockSpec((1, D), lambda i: (0, 0)),
            pl.BlockSpec((RPAD, 128), lambda i: (0, 0)),
            pl.BlockSpec((1, K), lambda i: (0, 0)),
        ),
        out_shape=(
            jax.ShapeDtypeStruct((1, 1), jnp.float32),
            jax.ShapeDtypeStruct((1, D), jnp.float32),
            jax.ShapeDtypeStruct((RPAD, 128), jnp.float32),
            jax.ShapeDtypeStruct((1, K), jnp.int32),
        ),
        scratch_shapes=[
            pltpu.VMEM((K, 128), jnp.float32),
            pltpu.VMEM((K, 128), jnp.int32),
            pltpu.VMEM((K, D), jnp.float32),
            pltpu.SemaphoreType.DMA,
        ],
        compiler_params=pltpu.CompilerParams(
            dimension_semantics=("arbitrary",)),
    )(x, W1, b1.reshape(1, A), W2, x,
      Wc1, bc1.reshape(1, H), Wc2, bc2.reshape(1, 1))

    logit = logit2d.reshape(())
    slide_embedding = emb2d.reshape(D)
    full_weights = fw2d.reshape(RPAD * 128)[:N]
    topk_idx = idx2d.reshape(K)
    return (logit, slide_embedding, full_weights, topk_idx)
